# baseline (speedup 1.0000x reference)
import functools

import jax
import jax.numpy as jnp
from jax import lax
from jax.experimental import pallas as pl
from jax.experimental.pallas import tpu as pltpu
from jax.experimental.pallas import tpu_sc as plsc

N_NODES = 10000
N_EDGES = 320000
D_FEAT = 128

NC = 2
NS = 16
NW = NC * NS
NP = 10240
SLAB = NP // NS

DH = D_FEAT // NC

B = 125
EPT = N_EDGES // NW
NB = EPT // B
NBC = 2 * NB
NRING = 8
LOOK = 5

_f32 = jnp.float32
_i32 = jnp.int32


def _zero_vec(ref, n):
    def body(i, carry):
        ref[pl.ds(i * 16, 16)] = jnp.zeros((16,), _f32)
        return carry
    lax.fori_loop(0, n // 16, body, 0)


def _zero_rows(ref, rows, cols):
    def body(r, carry):
        for k in range(cols // 16):
            ref[r, pl.ds(k * 16, 16)] = jnp.zeros((16,), _f32)
        return carry
    lax.fori_loop(0, rows, body, 0)


def _hist_body(src_hbm, dst_hbm, hs_hbm, hd_hbm,
               sidx, didx, ones, zv, hist_s, hist_d, sem):
    c = lax.axis_index("c")
    s = lax.axis_index("s")
    wid = c * NS + s

    pltpu.sync_copy(src_hbm.at[wid], sidx)
    pltpu.sync_copy(dst_hbm.at[wid], didx)
    for k in range(8):
        ones[pl.ds(k * 16, 16)] = jnp.ones((16,), _f32)
    _zero_vec(zv, SLAB)
    pltpu.sync_copy(zv, hist_s.at[pl.ds(s * SLAB, SLAB)])
    pltpu.sync_copy(zv, hist_d.at[pl.ds(s * SLAB, SLAB)])
    plsc.subcore_barrier()

    one_b = ones.at[pl.ds(0, B)]

    def body(j, carry):
        pltpu.async_copy(one_b, hist_s.at[sidx.at[j]], sem, add=True)
        pltpu.async_copy(one_b, hist_d.at[didx.at[j]], sem, add=True)
        return carry
    lax.fori_loop(0, NB, body, 0)

    def drain(j, carry):
        pltpu.make_async_copy(one_b, hist_s.at[sidx.at[j]], sem).wait()
        pltpu.make_async_copy(one_b, hist_d.at[didx.at[j]], sem).wait()
        return carry
    lax.fori_loop(0, NB, drain, 0)

    plsc.subcore_barrier()
    sl = pl.ds(s * SLAB, SLAB)
    pltpu.sync_copy(hist_s.at[sl], hs_hbm.at[c, sl])
    pltpu.sync_copy(hist_d.at[sl], hd_hbm.at[c, sl])


_hist = functools.partial(
    pl.kernel,
    out_type=(jax.ShapeDtypeStruct((NC, NP), _f32),
              jax.ShapeDtypeStruct((NC, NP), _f32)),
    mesh=plsc.VectorSubcoreMesh(core_axis_name="c", subcore_axis_name="s"),
    scratch_types=[
        pltpu.VMEM((NB, B), _i32),
        pltpu.VMEM((NB, B), _i32),
        pltpu.VMEM((128,), _f32),
        pltpu.VMEM((SLAB,), _f32),
        pltpu.VMEM_SHARED((NP,), _f32),
        pltpu.VMEM_SHARED((NP,), _f32),
        pltpu.SemaphoreType.DMA,
    ],
)(_hist_body)


def _scale_body(hist_ref, h_ref, feat_ref):
    deg = hist_ref[0, :N_NODES] + hist_ref[1, :N_NODES]
    ns = lax.rsqrt(jnp.maximum(deg, 1.0))
    scaled = h_ref[...] * ns[:, None]
    feat_ref[0] = scaled[:, :DH]
    feat_ref[1] = scaled[:, DH:]


def _scale(hist, h):
    return pl.pallas_call(
        _scale_body,
        out_shape=jax.ShapeDtypeStruct((NC, N_NODES, DH), _f32),
    )(hist, h)


def _agg_body(feat_hbm, src_hbm, dst_hbm, acc_hbm,
              sidx, didx, rows, zrow, accum, gsem, ssem):
    c = lax.axis_index("c")
    s = lax.axis_index("s")

    _zero_rows(zrow, 64, DH)
    for u in range(SLAB // 64):
        pltpu.sync_copy(zrow, accum.at[pl.ds(s * SLAB + u * 64, 64)])
    plsc.subcore_barrier()

    myfeat = feat_hbm.at[c]

    def gather(j, b):
        pltpu.async_copy(myfeat.at[sidx.at[j]], rows.at[b], gsem[b])

    def gather_wait(j, b):
        pltpu.make_async_copy(myfeat.at[sidx.at[j]], rows.at[b], gsem[b]).wait()

    def scat(j, b):
        pltpu.async_copy(rows.at[b], accum.at[didx.at[j]], ssem[b], add=True)

    def scat_wait(j, b):
        pltpu.make_async_copy(rows.at[b], accum.at[didx.at[j]], ssem[b]).wait()

    for h in range(2):
        pltpu.sync_copy(src_hbm.at[2 * s + h], sidx)
        pltpu.sync_copy(dst_hbm.at[2 * s + h], didx)
        for j in range(LOOK):
            gather(j, j)

        def body(t, carry):
            j0 = NRING * t
            for b in range(NRING):
                j = j0 + b
                gather_wait(j, b)
                scat(j, b)
                nxt = j + LOOK
                bn = (b + LOOK) % NRING

                @pl.when(nxt >= NRING)
                def _wait_prev():
                    pltpu.make_async_copy(
                        rows.at[bn], accum.at[didx.at[0]], ssem[bn]).wait()

                @pl.when(nxt < NB)
                def _prefetch():
                    gather(nxt, bn)
            return carry
        lax.fori_loop(0, NB // NRING, body, 0)

        for k in range(NRING - LOOK):
            scat_wait(0, (NB - (NRING - LOOK) + k) % NRING)

    plsc.subcore_barrier()
    sl = pl.ds(s * SLAB, SLAB)
    pltpu.sync_copy(accum.at[sl], acc_hbm.at[c, sl])


_aggregate = functools.partial(
    pl.kernel,
    out_type=jax.ShapeDtypeStruct((NC, NP, DH), _f32),
    mesh=plsc.VectorSubcoreMesh(core_axis_name="c", subcore_axis_name="s"),
    scratch_types=[
        pltpu.VMEM((NB, B), _i32),
        pltpu.VMEM((NB, B), _i32),
        pltpu.VMEM((NRING, B, DH), _f32),
        pltpu.VMEM((64, DH), _f32),
        pltpu.VMEM_SHARED((NP, DH), _f32),
        [pltpu.SemaphoreType.DMA] * NRING,
        [pltpu.SemaphoreType.DMA] * NRING,
    ],
    compiler_params=pltpu.CompilerParams(use_tc_tiling_on_sc=False),
)(_agg_body)


def _final_body(acc_ref, hist_ref, out_ref):
    deg = hist_ref[0, :N_NODES] + hist_ref[1, :N_NODES]
    nd = lax.rsqrt(jnp.maximum(deg, 1.0))
    out_ref[:, :DH] = acc_ref[0, :N_NODES, :] * nd[:, None]
    out_ref[:, DH:] = acc_ref[1, :N_NODES, :] * nd[:, None]


def _final(acc, hist):
    return pl.pallas_call(
        _final_body,
        out_shape=jax.ShapeDtypeStruct((N_NODES, D_FEAT), _f32),
    )(acc, hist)


def kernel(h, edge_index):
    src = edge_index[0].astype(_i32).reshape(NW, NB, B)
    dst = edge_index[1].astype(_i32).reshape(NW, NB, B)
    hist_s, hist_d = _hist(src, dst)
    feat = _scale(hist_s, h)
    acc = _aggregate(feat, src, dst)
    return _final(acc, hist_d)


# speedup vs baseline: 1.1278x; 1.1278x over previous
import functools

import jax
import jax.numpy as jnp
from jax import lax
from jax.experimental import pallas as pl
from jax.experimental.pallas import tpu as pltpu
from jax.experimental.pallas import tpu_sc as plsc

N_NODES = 10000
N_EDGES = 320000
D_FEAT = 128

NC = 2
NS = 16
NW = NC * NS
NP = 10240
SLAB = NP // NS

DH = D_FEAT // NC

B = 125
EPT = N_EDGES // NW
NB = EPT // B
NBC = 2 * NB
NRING = 8
LOOK = 6

_f32 = jnp.float32
_i32 = jnp.int32


def _zero_vec(ref, n):
    def body(i, carry):
        ref[pl.ds(i * 16, 16)] = jnp.zeros((16,), _f32)
        return carry
    lax.fori_loop(0, n // 16, body, 0)


def _zero_rows(ref, rows, cols):
    def body(r, carry):
        for k in range(cols // 16):
            ref[r, pl.ds(k * 16, 16)] = jnp.zeros((16,), _f32)
        return carry
    lax.fori_loop(0, rows, body, 0)


def _hist_body(src_hbm, dst_hbm, hs_hbm, hd_hbm,
               sidx, didx, ones, zv, hist_s, hist_d, sem):
    c = lax.axis_index("c")
    s = lax.axis_index("s")
    wid = c * NS + s

    pltpu.sync_copy(src_hbm.at[wid], sidx)
    pltpu.sync_copy(dst_hbm.at[wid], didx)
    for k in range(8):
        ones[pl.ds(k * 16, 16)] = jnp.ones((16,), _f32)
    _zero_vec(zv, SLAB)
    pltpu.sync_copy(zv, hist_s.at[pl.ds(s * SLAB, SLAB)])
    pltpu.sync_copy(zv, hist_d.at[pl.ds(s * SLAB, SLAB)])
    plsc.subcore_barrier()

    one_b = ones.at[pl.ds(0, B)]

    def body(j, carry):
        pltpu.async_copy(one_b, hist_s.at[sidx.at[j]], sem, add=True)
        pltpu.async_copy(one_b, hist_d.at[didx.at[j]], sem, add=True)
        return carry
    lax.fori_loop(0, NB, body, 0)

    def drain(j, carry):
        pltpu.make_async_copy(one_b, hist_s.at[sidx.at[j]], sem).wait()
        pltpu.make_async_copy(one_b, hist_d.at[didx.at[j]], sem).wait()
        return carry
    lax.fori_loop(0, NB, drain, 0)

    plsc.subcore_barrier()
    sl = pl.ds(s * SLAB, SLAB)
    pltpu.sync_copy(hist_s.at[sl], hs_hbm.at[c, sl])
    pltpu.sync_copy(hist_d.at[sl], hd_hbm.at[c, sl])


_hist = functools.partial(
    pl.kernel,
    out_type=(jax.ShapeDtypeStruct((NC, NP), _f32),
              jax.ShapeDtypeStruct((NC, NP), _f32)),
    mesh=plsc.VectorSubcoreMesh(core_axis_name="c", subcore_axis_name="s"),
    scratch_types=[
        pltpu.VMEM((NB, B), _i32),
        pltpu.VMEM((NB, B), _i32),
        pltpu.VMEM((128,), _f32),
        pltpu.VMEM((SLAB,), _f32),
        pltpu.VMEM_SHARED((NP,), _f32),
        pltpu.VMEM_SHARED((NP,), _f32),
        pltpu.SemaphoreType.DMA,
    ],
)(_hist_body)


def _scale_body(hist_ref, histd_ref, h_ref, feat_ref, ndv_ref):
    deg = hist_ref[0, :N_NODES] + hist_ref[1, :N_NODES]
    ns = lax.rsqrt(jnp.maximum(deg, 1.0))
    scaled = h_ref[...] * ns[:, None]
    feat_ref[0] = scaled[:, :DH]
    feat_ref[1] = scaled[:, DH:]
    degd = histd_ref[0] + histd_ref[1]
    ndv_ref[...] = lax.rsqrt(jnp.maximum(degd, 1.0))[None, :]


def _scale(hist, histd, h):
    return pl.pallas_call(
        _scale_body,
        out_shape=(jax.ShapeDtypeStruct((NC, N_NODES, DH), _f32),
                   jax.ShapeDtypeStruct((1, NP), _f32)),
    )(hist, histd, h)


CH = 80


def _agg_body(feat_hbm, src_hbm, dst_hbm, ndv_hbm, out_hbm,
              sidx, didx, rows, zrow, nd_v, accum, gsem, ssem):
    c = lax.axis_index("c")
    s = lax.axis_index("s")

    _zero_rows(zrow, 64, DH)
    for u in range(SLAB // 64):
        pltpu.sync_copy(zrow, accum.at[pl.ds(s * SLAB + u * 64, 64)])
    plsc.subcore_barrier()

    myfeat = feat_hbm.at[c]

    def gather(j, b):
        pltpu.async_copy(myfeat.at[sidx.at[j]], rows.at[b], gsem[b])

    def gather_wait(j, b):
        pltpu.make_async_copy(myfeat.at[sidx.at[j]], rows.at[b], gsem[b]).wait()

    def scat(j, b):
        pltpu.async_copy(rows.at[b], accum.at[didx.at[j]], ssem[b], add=True)

    def scat_wait(j, b):
        pltpu.make_async_copy(rows.at[b], accum.at[didx.at[j]], ssem[b]).wait()

    for h in range(2):
        pltpu.sync_copy(src_hbm.at[2 * s + h], sidx)
        pltpu.sync_copy(dst_hbm.at[2 * s + h], didx)
        for j in range(LOOK):
            gather(j, j)

        def body(t, carry):
            j0 = NRING * t
            for b in range(NRING):
                j = j0 + b
                gather_wait(j, b)
                scat(j, b)
                nxt = j + LOOK
                bn = (b + LOOK) % NRING

                @pl.when(nxt >= NRING)
                def _wait_prev():
                    pltpu.make_async_copy(
                        rows.at[bn], accum.at[didx.at[0]], ssem[bn]).wait()

                @pl.when(nxt < NB)
                def _prefetch():
                    gather(nxt, bn)
            return carry
        lax.fori_loop(0, NB // NRING, body, 0)

        for k in range(NRING - LOOK):
            scat_wait(0, (NB - (NRING - LOOK) + k) % NRING)

    plsc.subcore_barrier()

    pltpu.sync_copy(ndv_hbm.at[0, pl.ds(s * SLAB, SLAB)], nd_v)
    stage = rows.at[0]
    nchunks = jnp.where(s == NS - 1, (N_NODES - (NS - 1) * SLAB) // CH,
                        SLAB // CH)

    def chunk(u, carry):
        r0 = s * SLAB + u * CH
        pltpu.sync_copy(accum.at[pl.ds(r0, CH)], stage.at[pl.ds(0, CH)])

        for g in range(CH // 16):
            ndg = nd_v[pl.ds(u * CH + g * 16, 16)]
            for i in range(16):
                v = jnp.full((16,), ndg[i], _f32)
                for k in range(DH // 16):
                    sl = pl.ds(k * 16, 16)
                    stage[g * 16 + i, sl] = stage[g * 16 + i, sl] * v
        pltpu.sync_copy(stage.at[pl.ds(0, CH)],
                        out_hbm.at[pl.ds(r0, CH), pl.ds(c * DH, DH)])
        return carry
    lax.fori_loop(0, nchunks, chunk, 0)


_aggregate = functools.partial(
    pl.kernel,
    out_type=jax.ShapeDtypeStruct((N_NODES, D_FEAT), _f32),
    mesh=plsc.VectorSubcoreMesh(core_axis_name="c", subcore_axis_name="s"),
    scratch_types=[
        pltpu.VMEM((NB, B), _i32),
        pltpu.VMEM((NB, B), _i32),
        pltpu.VMEM((NRING, B, DH), _f32),
        pltpu.VMEM((64, DH), _f32),
        pltpu.VMEM((SLAB,), _f32),
        pltpu.VMEM_SHARED((NP, DH), _f32),
        [pltpu.SemaphoreType.DMA] * NRING,
        [pltpu.SemaphoreType.DMA] * NRING,
    ],
    compiler_params=pltpu.CompilerParams(use_tc_tiling_on_sc=False),
)(_agg_body)


def kernel(h, edge_index):
    src = edge_index[0].astype(_i32).reshape(NW, NB, B)
    dst = edge_index[1].astype(_i32).reshape(NW, NB, B)
    hist_s, hist_d = _hist(src, dst)
    feat, ndv = _scale(hist_s, hist_d, h)
    return _aggregate(feat, src, dst, ndv)


# speedup vs baseline: 1.4050x; 1.2458x over previous
import functools

import jax
import jax.numpy as jnp
from jax import lax
from jax.experimental import pallas as pl
from jax.experimental.pallas import tpu as pltpu
from jax.experimental.pallas import tpu_sc as plsc

N_NODES = 10000
N_EDGES = 320000
D_FEAT = 128

NC = 2
NS = 16
NW = NC * NS
NP = 10240
SLAB = NP // NS

DH = D_FEAT // NC

B = 125
EPT = N_EDGES // NW
NB = EPT // B
NBC = 2 * NB
NRING = 8
LOOK = 6

_f32 = jnp.float32
_i32 = jnp.int32


def _zero_vec(ref, n):
    def body(i, carry):
        ref[pl.ds(i * 16, 16)] = jnp.zeros((16,), _f32)
        return carry
    lax.fori_loop(0, n // 16, body, 0)


def _zero_rows(ref, rows, cols):
    lanes = 32 if ref.dtype == jnp.bfloat16 else 16

    def body(r, carry):
        for k in range(cols // lanes):
            ref[r, pl.ds(k * lanes, lanes)] = jnp.zeros((lanes,), ref.dtype)
        return carry
    lax.fori_loop(0, rows, body, 0)


def _hist_body(src_hbm, dst_hbm, hs_hbm, hd_hbm,
               sidx, didx, ones, zv, hist_s, hist_d, sem):
    c = lax.axis_index("c")
    s = lax.axis_index("s")
    wid = c * NS + s

    pltpu.sync_copy(src_hbm.at[wid], sidx)
    pltpu.sync_copy(dst_hbm.at[wid], didx)
    for k in range(8):
        ones[pl.ds(k * 16, 16)] = jnp.ones((16,), _f32)
    _zero_vec(zv, SLAB)
    pltpu.sync_copy(zv, hist_s.at[pl.ds(s * SLAB, SLAB)])
    pltpu.sync_copy(zv, hist_d.at[pl.ds(s * SLAB, SLAB)])
    plsc.subcore_barrier()

    one_b = ones.at[pl.ds(0, B)]

    def body(j, carry):
        pltpu.async_copy(one_b, hist_s.at[sidx.at[j]], sem, add=True)
        pltpu.async_copy(one_b, hist_d.at[didx.at[j]], sem, add=True)
        return carry
    lax.fori_loop(0, NB, body, 0)

    def drain(j, carry):
        pltpu.make_async_copy(one_b, hist_s.at[sidx.at[j]], sem).wait()
        pltpu.make_async_copy(one_b, hist_d.at[didx.at[j]], sem).wait()
        return carry
    lax.fori_loop(0, NB, drain, 0)

    plsc.subcore_barrier()
    sl = pl.ds(s * SLAB, SLAB)
    pltpu.sync_copy(hist_s.at[sl], hs_hbm.at[c, sl])
    pltpu.sync_copy(hist_d.at[sl], hd_hbm.at[c, sl])


_hist = functools.partial(
    pl.kernel,
    out_type=(jax.ShapeDtypeStruct((NC, NP), _f32),
              jax.ShapeDtypeStruct((NC, NP), _f32)),
    mesh=plsc.VectorSubcoreMesh(core_axis_name="c", subcore_axis_name="s"),
    scratch_types=[
        pltpu.VMEM((NB, B), _i32),
        pltpu.VMEM((NB, B), _i32),
        pltpu.VMEM((128,), _f32),
        pltpu.VMEM((SLAB,), _f32),
        pltpu.VMEM_SHARED((NP,), _f32),
        pltpu.VMEM_SHARED((NP,), _f32),
        pltpu.SemaphoreType.DMA,
    ],
)(_hist_body)


def _scale_body(hist_ref, histd_ref, h_ref, feat_ref, ndv_ref):
    deg = hist_ref[0, :N_NODES] + hist_ref[1, :N_NODES]
    ns = lax.rsqrt(jnp.maximum(deg, 1.0))
    scaled = h_ref[...] * ns[:, None]
    feat_ref[0] = scaled[:, :DH].astype(jnp.bfloat16)
    feat_ref[1] = scaled[:, DH:].astype(jnp.bfloat16)
    degd = histd_ref[0] + histd_ref[1]
    ndv_ref[...] = lax.rsqrt(jnp.maximum(degd, 1.0))[None, :]


def _scale(hist, histd, h):
    return pl.pallas_call(
        _scale_body,
        out_shape=(jax.ShapeDtypeStruct((NC, N_NODES, DH), jnp.bfloat16),
                   jax.ShapeDtypeStruct((1, NP), _f32)),
    )(hist, histd, h)


CH = 80


def _agg_body(feat_hbm, src_hbm, dst_hbm, ndv_hbm, out_hbm,
              sidx, didx, rows, zrow, nd_v, sout, accum, gsem, ssem):
    c = lax.axis_index("c")
    s = lax.axis_index("s")

    _zero_rows(zrow, 64, DH)
    for u in range(SLAB // 64):
        pltpu.sync_copy(zrow, accum.at[pl.ds(s * SLAB + u * 64, 64)])
    plsc.subcore_barrier()

    myfeat = feat_hbm.at[c]

    def gather(j, b):
        pltpu.async_copy(myfeat.at[sidx.at[j]], rows.at[b], gsem[b])

    def gather_wait(j, b):
        pltpu.make_async_copy(myfeat.at[sidx.at[j]], rows.at[b], gsem[b]).wait()

    def scat(j, b):
        pltpu.async_copy(rows.at[b], accum.at[didx.at[j]], ssem[b], add=True)

    def scat_wait(j, b):
        pltpu.make_async_copy(rows.at[b], accum.at[didx.at[j]], ssem[b]).wait()

    for h in range(2):
        pltpu.sync_copy(src_hbm.at[2 * s + h], sidx)
        pltpu.sync_copy(dst_hbm.at[2 * s + h], didx)
        for j in range(LOOK):
            gather(j, j)

        def body(t, carry):
            j0 = NRING * t
            for b in range(NRING):
                j = j0 + b
                gather_wait(j, b)
                scat(j, b)
                nxt = j + LOOK
                bn = (b + LOOK) % NRING

                @pl.when(nxt >= NRING)
                def _wait_prev():
                    pltpu.make_async_copy(
                        rows.at[bn], accum.at[didx.at[0]], ssem[bn]).wait()

                @pl.when(nxt < NB)
                def _prefetch():
                    gather(nxt, bn)
            return carry
        lax.fori_loop(0, NB // NRING, body, 0)

        for k in range(NRING - LOOK):
            scat_wait(0, (NB - (NRING - LOOK) + k) % NRING)

    plsc.subcore_barrier()

    pltpu.sync_copy(ndv_hbm.at[0, pl.ds(s * SLAB, SLAB)], nd_v)
    stage = rows.at[0]
    nchunks = jnp.where(s == NS - 1, (N_NODES - (NS - 1) * SLAB) // CH,
                        SLAB // CH)

    def chunk(u, carry):
        r0 = s * SLAB + u * CH
        pltpu.sync_copy(accum.at[pl.ds(r0, CH)], stage.at[pl.ds(0, CH)])
        for g in range(CH // 16):
            ndg = nd_v[pl.ds(u * CH + g * 16, 16)]
            for i in range(16):
                v = jnp.full((16,), ndg[i], _f32)
                r = g * 16 + i
                for g2 in range(DH // 32):
                    a, bb = plsc.unpack(stage[r, pl.ds(g2 * 32, 32)],
                                        format=plsc.PackFormat.INTERLEAVED)
                    sout[r, pl.ds(g2 * 32, 16)] = a * v
                    sout[r, pl.ds(g2 * 32 + 16, 16)] = bb * v
        pltpu.sync_copy(sout, out_hbm.at[pl.ds(r0, CH), pl.ds(c * DH, DH)])
        return carry
    lax.fori_loop(0, nchunks, chunk, 0)


_aggregate = functools.partial(
    pl.kernel,
    out_type=jax.ShapeDtypeStruct((N_NODES, D_FEAT), _f32),
    mesh=plsc.VectorSubcoreMesh(core_axis_name="c", subcore_axis_name="s"),
    scratch_types=[
        pltpu.VMEM((NB, B), _i32),
        pltpu.VMEM((NB, B), _i32),
        pltpu.VMEM((NRING, B, DH), jnp.bfloat16),
        pltpu.VMEM((64, DH), jnp.bfloat16),
        pltpu.VMEM((SLAB,), _f32),
        pltpu.VMEM((CH, DH), _f32),
        pltpu.VMEM_SHARED((NP, DH), jnp.bfloat16),
        [pltpu.SemaphoreType.DMA] * NRING,
        [pltpu.SemaphoreType.DMA] * NRING,
    ],
    compiler_params=pltpu.CompilerParams(use_tc_tiling_on_sc=False,
                                         needs_layout_passes=False),
)(_agg_body)


_PERM = []
for _c in range(NC):
    for _g2 in range(DH // 32):
        for _i in range(16):
            _PERM.extend([_c * DH + _g2 * 32 + _i,
                          _c * DH + _g2 * 32 + 16 + _i])


def kernel(h, edge_index):
    src = edge_index[0].astype(_i32).reshape(NW, NB, B)
    dst = edge_index[1].astype(_i32).reshape(NW, NB, B)
    hp = h[:, jnp.array(_PERM, dtype=jnp.int32)]
    hist_s, hist_d = _hist(src, dst)
    feat, ndv = _scale(hist_s, hist_d, hp)
    return _aggregate(feat, src, dst, ndv)


# speedup vs baseline: 1.4057x; 1.0005x over previous
import functools

import jax
import jax.numpy as jnp
from jax import lax
from jax.experimental import pallas as pl
from jax.experimental.pallas import tpu as pltpu
from jax.experimental.pallas import tpu_sc as plsc

N_NODES = 10000
N_EDGES = 320000
D_FEAT = 128

NC = 2
NS = 16
NW = NC * NS
NP = 10240
SLAB = NP // NS

DH = D_FEAT // NC

B = 125
EPT = N_EDGES // NW
NB = EPT // B
NBC = 2 * NB
NRING = 8
LOOK = 6

_f32 = jnp.float32
_i32 = jnp.int32
FIX = 1024.0


def _zero_vec(ref, n):
    def body(i, carry):
        ref[pl.ds(i * 16, 16)] = jnp.zeros((16,), _f32)
        return carry
    lax.fori_loop(0, n // 16, body, 0)


def _zero_rows(ref, rows, cols):
    lanes = 32 if jnp.dtype(ref.dtype).itemsize == 2 else 16

    def body(r, carry):
        for k in range(cols // lanes):
            ref[r, pl.ds(k * lanes, lanes)] = jnp.zeros((lanes,), ref.dtype)
        return carry
    lax.fori_loop(0, rows, body, 0)


def _hist_body(src_hbm, dst_hbm, hs_hbm, hd_hbm,
               sidx, didx, ones, zv, hist_s, hist_d, sem):
    c = lax.axis_index("c")
    s = lax.axis_index("s")
    wid = c * NS + s

    pltpu.sync_copy(src_hbm.at[wid], sidx)
    pltpu.sync_copy(dst_hbm.at[wid], didx)
    for k in range(8):
        ones[pl.ds(k * 16, 16)] = jnp.ones((16,), _f32)
    _zero_vec(zv, SLAB)
    pltpu.sync_copy(zv, hist_s.at[pl.ds(s * SLAB, SLAB)])
    pltpu.sync_copy(zv, hist_d.at[pl.ds(s * SLAB, SLAB)])
    plsc.subcore_barrier()

    one_b = ones.at[pl.ds(0, B)]

    def body(j, carry):
        pltpu.async_copy(one_b, hist_s.at[sidx.at[j]], sem, add=True)
        pltpu.async_copy(one_b, hist_d.at[didx.at[j]], sem, add=True)
        return carry
    lax.fori_loop(0, NB, body, 0)

    def drain(j, carry):
        pltpu.make_async_copy(one_b, hist_s.at[sidx.at[j]], sem).wait()
        pltpu.make_async_copy(one_b, hist_d.at[didx.at[j]], sem).wait()
        return carry
    lax.fori_loop(0, NB, drain, 0)

    plsc.subcore_barrier()
    sl = pl.ds(s * SLAB, SLAB)
    pltpu.sync_copy(hist_s.at[sl], hs_hbm.at[c, sl])
    pltpu.sync_copy(hist_d.at[sl], hd_hbm.at[c, sl])


_hist = functools.partial(
    pl.kernel,
    out_type=(jax.ShapeDtypeStruct((NC, NP), _f32),
              jax.ShapeDtypeStruct((NC, NP), _f32)),
    mesh=plsc.VectorSubcoreMesh(core_axis_name="c", subcore_axis_name="s"),
    scratch_types=[
        pltpu.VMEM((NB, B), _i32),
        pltpu.VMEM((NB, B), _i32),
        pltpu.VMEM((128,), _f32),
        pltpu.VMEM((SLAB,), _f32),
        pltpu.VMEM_SHARED((NP,), _f32),
        pltpu.VMEM_SHARED((NP,), _f32),
        pltpu.SemaphoreType.DMA,
    ],
)(_hist_body)


def _scale_body(hist_ref, histd_ref, h_ref, feat_ref, ndv_ref):
    deg = hist_ref[0, :N_NODES] + hist_ref[1, :N_NODES]
    ns = lax.rsqrt(jnp.maximum(deg, 1.0))
    scaled = h_ref[...] * ns[:, None]
    q = jnp.round(scaled * FIX)
    feat_ref[0] = q[:, :DH].astype(jnp.int16)
    feat_ref[1] = q[:, DH:].astype(jnp.int16)
    degd = histd_ref[0] + histd_ref[1]
    ndv_ref[...] = (lax.rsqrt(jnp.maximum(degd, 1.0)) / FIX)[None, :]


def _scale(hist, histd, h):
    return pl.pallas_call(
        _scale_body,
        out_shape=(jax.ShapeDtypeStruct((NC, N_NODES, DH), jnp.int16),
                   jax.ShapeDtypeStruct((1, NP), _f32)),
    )(hist, histd, h)


CH = 80


def _agg_body(feat_hbm, src_hbm, dst_hbm, ndv_hbm, out_hbm,
              sidx, didx, rows, zrow, nd_v, sout, accum, gsem, ssem):
    c = lax.axis_index("c")
    s = lax.axis_index("s")

    _zero_rows(zrow, 64, DH)
    for u in range(SLAB // 64):
        pltpu.sync_copy(zrow, accum.at[pl.ds(s * SLAB + u * 64, 64)])
    plsc.subcore_barrier()

    myfeat = feat_hbm.at[c]

    def gather(j, b):
        pltpu.async_copy(myfeat.at[sidx.at[j]], rows.at[b], gsem[b])

    def gather_wait(j, b):
        pltpu.make_async_copy(myfeat.at[sidx.at[j]], rows.at[b], gsem[b]).wait()

    def scat(j, b):
        pltpu.async_copy(rows.at[b], accum.at[didx.at[j]], ssem[b], add=True)

    def scat_wait(j, b):
        pltpu.make_async_copy(rows.at[b], accum.at[didx.at[j]], ssem[b]).wait()

    for h in range(2):
        pltpu.sync_copy(src_hbm.at[2 * s + h], sidx)
        pltpu.sync_copy(dst_hbm.at[2 * s + h], didx)
        for j in range(LOOK):
            gather(j, j)

        def body(t, carry):
            j0 = NRING * t
            for b in range(NRING):
                j = j0 + b
                gather_wait(j, b)
                scat(j, b)
                nxt = j + LOOK
                bn = (b + LOOK) % NRING

                @pl.when(nxt >= NRING)
                def _wait_prev():
                    pltpu.make_async_copy(
                        rows.at[bn], accum.at[didx.at[0]], ssem[bn]).wait()

                @pl.when(nxt < NB)
                def _prefetch():
                    gather(nxt, bn)
            return carry
        lax.fori_loop(0, NB // NRING, body, 0)

        for k in range(NRING - LOOK):
            scat_wait(0, (NB - (NRING - LOOK) + k) % NRING)

    plsc.subcore_barrier()

    pltpu.sync_copy(ndv_hbm.at[0, pl.ds(s * SLAB, SLAB)], nd_v)
    stage = rows.at[0]
    nchunks = jnp.where(s == NS - 1, (N_NODES - (NS - 1) * SLAB) // CH,
                        SLAB // CH)

    def chunk(u, carry):
        r0 = s * SLAB + u * CH
        pltpu.sync_copy(accum.at[pl.ds(r0, CH)], stage.at[pl.ds(0, CH)])
        for g in range(CH // 16):
            ndg = nd_v[pl.ds(u * CH + g * 16, 16)]
            for i in range(16):
                v = jnp.full((16,), ndg[i], _f32)
                r = g * 16 + i
                for g2 in range(DH // 32):
                    a, bb = plsc.unpack(stage[r, pl.ds(g2 * 32, 32)],
                                        format=plsc.PackFormat.INTERLEAVED,
                                        preferred_element_type=_i32)
                    sout[r, pl.ds(g2 * 32, 16)] = a.astype(_f32) * v
                    sout[r, pl.ds(g2 * 32 + 16, 16)] = bb.astype(_f32) * v
        pltpu.sync_copy(sout, out_hbm.at[pl.ds(r0, CH), pl.ds(c * DH, DH)])
        return carry
    lax.fori_loop(0, nchunks, chunk, 0)


_aggregate = functools.partial(
    pl.kernel,
    out_type=jax.ShapeDtypeStruct((N_NODES, D_FEAT), _f32),
    mesh=plsc.VectorSubcoreMesh(core_axis_name="c", subcore_axis_name="s"),
    scratch_types=[
        pltpu.VMEM((NB, B), _i32),
        pltpu.VMEM((NB, B), _i32),
        pltpu.VMEM((NRING, B, DH), jnp.int16),
        pltpu.VMEM((64, DH), jnp.int16),
        pltpu.VMEM((SLAB,), _f32),
        pltpu.VMEM((CH, DH), _f32),
        pltpu.VMEM_SHARED((NP, DH), jnp.int16),
        [pltpu.SemaphoreType.DMA] * NRING,
        [pltpu.SemaphoreType.DMA] * NRING,
    ],
    compiler_params=pltpu.CompilerParams(use_tc_tiling_on_sc=False,
                                         needs_layout_passes=False),
)(_agg_body)


_PERM = []
for _c in range(NC):
    for _g2 in range(DH // 32):
        for _i in range(16):
            _PERM.extend([_c * DH + _g2 * 32 + _i,
                          _c * DH + _g2 * 32 + 16 + _i])


def kernel(h, edge_index):
    src = edge_index[0].astype(_i32).reshape(NW, NB, B)
    dst = edge_index[1].astype(_i32).reshape(NW, NB, B)
    hp = h[:, jnp.array(_PERM, dtype=jnp.int32)]
    hist_s, hist_d = _hist(src, dst)
    feat, ndv = _scale(hist_s, hist_d, hp)
    return _aggregate(feat, src, dst, ndv)
